# Initial kernel scaffold; baseline (speedup 1.0000x reference)
import functools

import jax
import jax.numpy as jnp
from jax import lax
from jax.experimental import pallas as pl
from jax.experimental.pallas import tpu as pltpu
from jax.experimental.pallas import tpu_sc as plsc

N = 10000
E = 320000
D = 128
R = 8
NBASES = 30

NC = 2
NS = 16
NW = NC * NS
K = 128
EPW = -(-E // (NW * K)) * K
EP = NW * EPW
NBLK = EPW // K

NPAD = 10240
RPT = NPAD // NS
CBINS = 80128
CPT = CBINS // NS

f32 = jnp.float32
i32 = jnp.int32


def _mesh():
    return plsc.VectorSubcoreMesh(
        core_axis_name="c", subcore_axis_name="s", num_cores=NC, num_subcores=NS
    )


def _worker_id():
    return lax.axis_index("s") * NC + lax.axis_index("c")


def _count_sc(dst_p, et_p):
    @functools.partial(
        pl.kernel,
        out_type=jax.ShapeDtypeStruct((NC * CBINS,), f32),
        mesh=_mesh(),
        scratch_types=[
            pltpu.VMEM((K,), i32),
            pltpu.VMEM((K,), i32),
            pltpu.VMEM((K,), i32),
            pltpu.VMEM((K,), f32),
            pltpu.VMEM((CPT,), f32),
            pltpu.VMEM_SHARED((CBINS,), f32),
        ],
    )
    def k(dst_hbm, et_hbm, out_hbm, dbuf, tbuf, keybuf, ones, zbuf, cnt_sh):
        cid = lax.axis_index("c")
        sid = lax.axis_index("s")
        wid = _worker_id()

        @pl.loop(0, CPT // 16)
        def _z(i):
            zbuf[pl.ds(i * 16, 16)] = jnp.zeros((16,), f32)

        @pl.loop(0, K // 16)
        def _o(i):
            ones[pl.ds(i * 16, 16)] = jnp.ones((16,), f32)

        pltpu.sync_copy(zbuf, cnt_sh.at[pl.ds(sid * CPT, CPT)])
        plsc.subcore_barrier()

        @pl.loop(0, NBLK)
        def _b(b):
            base = wid * EPW + b * K
            pltpu.sync_copy(dst_hbm.at[pl.ds(base, K)], dbuf)
            pltpu.sync_copy(et_hbm.at[pl.ds(base, K)], tbuf)

            @pl.loop(0, K // 16)
            def _c(i):
                sl = pl.ds(i * 16, 16)
                keybuf[sl] = dbuf[sl] * R + tbuf[sl]

            pltpu.sync_copy(ones, cnt_sh.at[keybuf], add=True)

        plsc.subcore_barrier()
        pltpu.sync_copy(
            cnt_sh.at[pl.ds(sid * CPT, CPT)],
            out_hbm.at[pl.ds(cid * CBINS + sid * CPT, CPT)],
        )

    return k(dst_p, et_p)


def _rgcn_sc(src_p, dst_p, et_p, c0, c1, yflat):
    @functools.partial(
        pl.kernel,
        out_type=jax.ShapeDtypeStruct((NC * NPAD, D), f32),
        mesh=_mesh(),
        scratch_types=[
            pltpu.VMEM((K,), i32),
            pltpu.VMEM((K,), i32),
            pltpu.VMEM((K,), i32),
            pltpu.VMEM((K,), i32),
            pltpu.VMEM((K,), i32),
            pltpu.VMEM((K,), f32),
            pltpu.VMEM((K,), f32),
            pltpu.VMEM((K,), f32),
            pltpu.VMEM((K, D), f32),
            pltpu.VMEM_SHARED((NPAD, D), f32),
        ],
    )
    def k(src_hbm, dst_hbm, et_hbm, c0_hbm, c1_hbm, y_hbm, out_hbm,
          sbuf, dbuf, tbuf, ibuf, kbuf, c0b, c1b, scb, rows, acc_sh):
        cid = lax.axis_index("c")
        sid = lax.axis_index("s")
        wid = _worker_id()

        @pl.loop(0, K)
        def _zr(e):
            for c in range(D // 16):
                rows[e, pl.ds(c * 16, 16)] = jnp.zeros((16,), f32)

        for j in range(RPT // K):
            pltpu.sync_copy(rows, acc_sh.at[pl.ds(sid * RPT + j * K, K)])
        plsc.subcore_barrier()

        @pl.loop(0, NBLK)
        def _b(b):
            base = wid * EPW + b * K
            pltpu.sync_copy(src_hbm.at[pl.ds(base, K)], sbuf)
            pltpu.sync_copy(dst_hbm.at[pl.ds(base, K)], dbuf)
            pltpu.sync_copy(et_hbm.at[pl.ds(base, K)], tbuf)

            @pl.loop(0, K // 16)
            def _i(i):
                sl = pl.ds(i * 16, 16)
                ibuf[sl] = tbuf[sl] * N + sbuf[sl]
                kbuf[sl] = dbuf[sl] * R + tbuf[sl]

            pltpu.sync_copy(c0_hbm.at[kbuf], c0b)
            pltpu.sync_copy(c1_hbm.at[kbuf], c1b)

            @pl.loop(0, K // 16)
            def _s(i):
                sl = pl.ds(i * 16, 16)
                scb[sl] = 1.0 / jnp.maximum(c0b[sl] + c1b[sl], 1.0)

            pltpu.sync_copy(y_hbm.at[ibuf], rows)

            @pl.loop(0, K)
            def _e(e):
                spl = plsc.load_gather(scb, [jnp.full((16,), e, i32)])
                for c in range(D // 16):
                    sl = pl.ds(c * 16, 16)
                    rows[e, sl] = rows[e, sl] * spl

            pltpu.sync_copy(rows, acc_sh.at[dbuf], add=True)

        plsc.subcore_barrier()
        for j in range(RPT // K):
            off = sid * RPT + j * K
            pltpu.sync_copy(acc_sh.at[pl.ds(off, K)],
                            out_hbm.at[pl.ds(cid * NPAD + off, K)])

    return k(src_p, dst_p, et_p, c0, c1, yflat)


def _gconv_sc(src_p, dst_p, z):
    @functools.partial(
        pl.kernel,
        out_type=jax.ShapeDtypeStruct((NC * NPAD, D), f32),
        mesh=_mesh(),
        scratch_types=[
            pltpu.VMEM((K,), i32),
            pltpu.VMEM((K,), i32),
            pltpu.VMEM((K, D), f32),
            pltpu.VMEM_SHARED((NPAD, D), f32),
        ],
    )
    def k(src_hbm, dst_hbm, z_hbm, out_hbm, sbuf, dbuf, rows, acc_sh):
        cid = lax.axis_index("c")
        sid = lax.axis_index("s")
        wid = _worker_id()

        @pl.loop(0, K)
        def _zr(e):
            for c in range(D // 16):
                rows[e, pl.ds(c * 16, 16)] = jnp.zeros((16,), f32)

        for j in range(RPT // K):
            pltpu.sync_copy(rows, acc_sh.at[pl.ds(sid * RPT + j * K, K)])
        plsc.subcore_barrier()

        @pl.loop(0, NBLK)
        def _b(b):
            base = wid * EPW + b * K
            pltpu.sync_copy(src_hbm.at[pl.ds(base, K)], sbuf)
            pltpu.sync_copy(dst_hbm.at[pl.ds(base, K)], dbuf)
            pltpu.sync_copy(z_hbm.at[sbuf], rows)
            pltpu.sync_copy(rows, acc_sh.at[dbuf], add=True)

        plsc.subcore_barrier()
        for j in range(RPT // K):
            off = sid * RPT + j * K
            pltpu.sync_copy(acc_sh.at[pl.ds(off, K)],
                            out_hbm.at[pl.ds(cid * NPAD + off, K)])

    return k(src_p, dst_p, z)


def _weight_tc(comp, bases2d):
    def body(c_ref, b_ref, w_ref):
        w_ref[...] = jnp.dot(c_ref[...], b_ref[...],
                             preferred_element_type=f32)

    return pl.pallas_call(
        body,
        out_shape=jax.ShapeDtypeStruct((R, D * D), f32),
    )(comp, bases2d)


def _y_tc(x, w3):
    NT = 10
    BN = N // NT

    def body(x_ref, w_ref, y_ref):
        y_ref[0] = jnp.dot(x_ref[...], w_ref[0], preferred_element_type=f32)

    return pl.pallas_call(
        body,
        grid=(R, NT),
        in_specs=[
            pl.BlockSpec((BN, D), lambda r, n: (n, 0)),
            pl.BlockSpec((1, D, D), lambda r, n: (r, 0, 0)),
        ],
        out_specs=pl.BlockSpec((1, BN, D), lambda r, n: (r, n, 0)),
        out_shape=jax.ShapeDtypeStruct((R, N, D), f32),
    )(x, w3)


def _fuse_tc(part, xin, w, bias, w2=None):
    NT = 10
    BN = N // NT
    specs = [
        pl.BlockSpec((NC, BN, D), lambda n: (0, n, 0)),
        pl.BlockSpec((BN, D), lambda n: (n, 0)),
        pl.BlockSpec((D, D), lambda n: (0, 0)),
        pl.BlockSpec((1, D), lambda n: (0, 0)),
    ]
    ospec = pl.BlockSpec((BN, D), lambda n: (n, 0))
    oshape = jax.ShapeDtypeStruct((N, D), f32)

    if w2 is None:
        def body(p_ref, x_ref, w_ref, b_ref, o_ref):
            o_ref[...] = (p_ref[0] + p_ref[1] + b_ref[...]
                          + jnp.dot(x_ref[...], w_ref[...],
                                    preferred_element_type=f32))

        return pl.pallas_call(
            body, grid=(NT,), in_specs=specs, out_specs=ospec,
            out_shape=oshape,
        )(part, xin, w, bias)

    def body(p_ref, x_ref, w_ref, b_ref, w2_ref, o_ref, z_ref):
        o = (p_ref[0] + p_ref[1] + b_ref[...]
             + jnp.dot(x_ref[...], w_ref[...], preferred_element_type=f32))
        o_ref[...] = o
        z_ref[...] = jnp.dot(o, w2_ref[...], preferred_element_type=f32)

    return pl.pallas_call(
        body, grid=(NT,),
        in_specs=specs + [pl.BlockSpec((D, D), lambda n: (0, 0))],
        out_specs=(ospec, ospec),
        out_shape=(oshape, oshape),
    )(part, xin, w, bias, w2)


def kernel(node_features, edge_index, edge_norm, edge_type, comp, bases,
           root1, bias1, w_rel, w_root, bias2):
    del edge_norm
    x = node_features
    src = edge_index[0]
    dst = edge_index[1]
    pad = EP - E
    src_p = jnp.concatenate([src, jnp.zeros((pad,), i32)])
    dst_p = jnp.concatenate([dst, jnp.full((pad,), N, i32)])
    et_p = jnp.concatenate([edge_type, jnp.zeros((pad,), i32)])

    w2d = _weight_tc(comp, bases.reshape(NBASES, D * D))
    y = _y_tc(x, w2d.reshape(R, D, D)).reshape(R * N, D)

    cnt = _count_sc(dst_p, et_p)
    c0 = cnt[:CBINS]
    c1 = cnt[CBINS:]

    outp = _rgcn_sc(src_p, dst_p, et_p, c0, c1, y)
    outp2 = outp.reshape(NC, NPAD, D)[:, :N]
    x1, z = _fuse_tc(outp2, x, root1, bias1.reshape(1, D), w2=w_rel)

    aggp = _gconv_sc(src_p, dst_p, z)
    aggp2 = aggp.reshape(NC, NPAD, D)[:, :N]
    x2 = _fuse_tc(aggp2, x1, w_root, bias2.reshape(1, D))
    return x2


# speedup vs baseline: 8.8684x; 8.8684x over previous
import functools

import jax
import jax.numpy as jnp
from jax import lax
from jax.experimental import pallas as pl
from jax.experimental.pallas import tpu as pltpu
from jax.experimental.pallas import tpu_sc as plsc

N = 10000
E = 320000
D = 128
R = 8
NBASES = 30

NC = 2
NS = 16
NW = NC * NS
K = 64
DEPTH = 4
EPW = -(-E // (NW * K * DEPTH)) * K * DEPTH
EP = NW * EPW
NBLK = EPW // K
NHALF = NBLK // 2
ROUNDS = NBLK // DEPTH
G = EP // K

NPAD = 10240
RPT = NPAD // NS
CBINS = 80128
CPT = CBINS // NS

f32 = jnp.float32
i32 = jnp.int32


def _mesh():
    return plsc.VectorSubcoreMesh(
        core_axis_name="c", subcore_axis_name="s", num_cores=NC, num_subcores=NS
    )


def _worker_id():
    return lax.axis_index("s") * NC + lax.axis_index("c")


def _splat(vec16, lane):
    idx = jnp.full((16, 1), lane, i32)
    return lax.gather(
        vec16, idx,
        dimension_numbers=lax.GatherDimensionNumbers(
            offset_dims=(), collapsed_slice_dims=(0,), start_index_map=(0,)),
        slice_sizes=(1,),
        mode=lax.GatherScatterMode.PROMISE_IN_BOUNDS,
    )


def _count_sc(comb):
    @functools.partial(
        pl.kernel,
        out_type=jax.ShapeDtypeStruct((NC * CBINS,), f32),
        mesh=_mesh(),
        scratch_types=[
            pltpu.VMEM((K,), i32),
            pltpu.VMEM((K,), i32),
            pltpu.VMEM((K,), i32),
            pltpu.VMEM((K,), i32),
            pltpu.VMEM((K,), f32),
            pltpu.VMEM((CPT,), f32),
            pltpu.VMEM_SHARED((CBINS,), f32),
            pltpu.SemaphoreType.DMA,
            pltpu.SemaphoreType.DMA,
            pltpu.SemaphoreType.DMA,
            pltpu.SemaphoreType.DMA,
        ],
    )
    def k(comb_hbm, out_hbm, cbufA, cbufB, kbufA, kbufB, ones, zbuf, cnt_sh,
          semcA, semcB, semsA, semsB):
        cid = lax.axis_index("c")
        sid = lax.axis_index("s")
        wid = _worker_id()

        @pl.loop(0, CPT // 16)
        def _z(i):
            zbuf[pl.ds(i * 16, 16)] = jnp.zeros((16,), f32)

        @pl.loop(0, K // 16)
        def _o(i):
            ones[pl.ds(i * 16, 16)] = jnp.ones((16,), f32)

        pltpu.sync_copy(zbuf, cnt_sh.at[pl.ds(sid * CPT, CPT)])
        plsc.subcore_barrier()

        def blk_slice(b):
            return comb_hbm.at[wid * NBLK + b]

        pltpu.async_copy(blk_slice(0), cbufA, semcA)
        pltpu.async_copy(blk_slice(1), cbufB, semcB)

        @pl.loop(0, NHALF)
        def _i2(i2):
            for off, cbuf, kbuf, semc, sems in (
                    (0, cbufA, kbufA, semcA, semsA),
                    (1, cbufB, kbufB, semcB, semsB)):
                b = i2 * 2 + off
                pltpu.make_async_copy(blk_slice(b), cbuf, semc).wait()

                @pl.when(i2 > 0)
                def _w():
                    pltpu.make_async_copy(ones, cnt_sh.at[kbuf], sems).wait()

                @pl.loop(0, K // 16)
                def _c(i):
                    sl = pl.ds(i * 16, 16)
                    code = cbuf[sl]
                    kbuf[sl] = ((code >> 14) & 0x3FFF) * R + (code >> 28)

                @pl.when(i2 < NHALF - 1)
                def _p():
                    pltpu.async_copy(blk_slice(b + 2), cbuf, semc)

                pltpu.async_copy(ones, cnt_sh.at[kbuf], sems, add=True)

        pltpu.make_async_copy(ones, cnt_sh.at[kbufA], semsA).wait()
        pltpu.make_async_copy(ones, cnt_sh.at[kbufB], semsB).wait()
        plsc.subcore_barrier()
        pltpu.sync_copy(cnt_sh.at[pl.ds(sid * CPT, CPT)], zbuf)
        pltpu.sync_copy(zbuf, out_hbm.at[pl.ds(cid * CBINS + sid * CPT, CPT)])

    return k(comb)


def _rgcn_sc(comb, inv, yflat):
    @functools.partial(
        pl.kernel,
        out_type=jax.ShapeDtypeStruct((NC * NPAD, D), f32),
        mesh=_mesh(),
        scratch_types=(
            [pltpu.VMEM((K,), i32)] * DEPTH
            + [pltpu.VMEM((K,), i32)] * DEPTH
            + [pltpu.VMEM((K,), i32)] * DEPTH
            + [pltpu.VMEM((K,), i32)] * DEPTH
            + [pltpu.VMEM((K,), f32)] * DEPTH
            + [pltpu.VMEM((K, D), f32)] * DEPTH
            + [pltpu.VMEM_SHARED((NPAD, D), f32)]
            + [pltpu.SemaphoreType.DMA] * (3 * DEPTH)
        ),
    )
    def k(comb_hbm, inv_hbm, y_hbm, out_hbm, *scr):
        cbuf = scr[0:DEPTH]
        ibuf = scr[DEPTH:2 * DEPTH]
        kbuf = scr[2 * DEPTH:3 * DEPTH]
        dbuf = scr[3 * DEPTH:4 * DEPTH]
        scb = scr[4 * DEPTH:5 * DEPTH]
        rows = scr[5 * DEPTH:6 * DEPTH]
        acc_sh = scr[6 * DEPTH]
        semc = scr[6 * DEPTH + 1:6 * DEPTH + 1 + DEPTH]
        semg = scr[6 * DEPTH + 1 + DEPTH:6 * DEPTH + 1 + 2 * DEPTH]
        sems = scr[6 * DEPTH + 1 + 2 * DEPTH:6 * DEPTH + 1 + 3 * DEPTH]

        cid = lax.axis_index("c")
        sid = lax.axis_index("s")
        wid = _worker_id()

        for e in range(K):
            for c in range(D // 16):
                rows[0][e, pl.ds(c * 16, 16)] = jnp.zeros((16,), f32)
        for j in range(RPT // K):
            pltpu.sync_copy(rows[0], acc_sh.at[pl.ds(sid * RPT + j * K, K)])
        plsc.subcore_barrier()

        def blk_slice(b):
            return comb_hbm.at[wid * NBLK + b]

        for d in range(DEPTH):
            pltpu.async_copy(blk_slice(d), cbuf[d], semc[d])

        def stage_front(i, d):
            b = i * DEPTH + d

            @pl.when(i > 0)
            def _ws():
                pltpu.make_async_copy(rows[d], acc_sh.at[dbuf[d]],
                                      sems[d]).wait()

            pltpu.make_async_copy(blk_slice(b), cbuf[d], semc[d]).wait()

            @pl.loop(0, K // 16)
            def _ix(j):
                sl = pl.ds(j * 16, 16)
                code = cbuf[d][sl]
                s16 = code & 0x3FFF
                d16 = (code >> 14) & 0x3FFF
                t16 = code >> 28
                ibuf[d][sl] = t16 * N + s16
                kbuf[d][sl] = d16 * R + t16
                dbuf[d][sl] = d16

            pltpu.async_copy(inv_hbm.at[kbuf[d]], scb[d], semg[d])
            pltpu.async_copy(y_hbm.at[ibuf[d]], rows[d], semg[d])

            @pl.when(i < ROUNDS - 1)
            def _p():
                pltpu.async_copy(blk_slice(b + DEPTH), cbuf[d], semc[d])

        def stage_back(d):
            pltpu.make_async_copy(inv_hbm.at[kbuf[d]], scb[d], semg[d]).wait()
            pltpu.make_async_copy(y_hbm.at[ibuf[d]], rows[d], semg[d]).wait()
            for gch in range(K // 16):
                sc16 = scb[d][pl.ds(gch * 16, 16)]
                for l in range(16):
                    spl = _splat(sc16, l)
                    e = gch * 16 + l
                    for c in range(D // 16):
                        sl = pl.ds(c * 16, 16)
                        rows[d][e, sl] = rows[d][e, sl] * spl
            pltpu.async_copy(rows[d], acc_sh.at[dbuf[d]], sems[d], add=True)

        @pl.loop(0, ROUNDS)
        def _r(i):
            for d in range(DEPTH):
                stage_front(i, d)
            for d in range(DEPTH):
                stage_back(d)

        for d in range(DEPTH):
            pltpu.make_async_copy(rows[d], acc_sh.at[dbuf[d]], sems[d]).wait()
        plsc.subcore_barrier()
        for j in range(RPT // K):
            off = sid * RPT + j * K
            pltpu.sync_copy(acc_sh.at[pl.ds(off, K)], rows[0])
            pltpu.sync_copy(rows[0], out_hbm.at[pl.ds(cid * NPAD + off, K)])

    return k(comb, inv, yflat)


def _gconv_sc(comb, z):
    @functools.partial(
        pl.kernel,
        out_type=jax.ShapeDtypeStruct((NC * NPAD, D), f32),
        mesh=_mesh(),
        scratch_types=(
            [pltpu.VMEM((K,), i32)] * DEPTH
            + [pltpu.VMEM((K,), i32)] * DEPTH
            + [pltpu.VMEM((K,), i32)] * DEPTH
            + [pltpu.VMEM((K, D), f32)] * DEPTH
            + [pltpu.VMEM_SHARED((NPAD, D), f32)]
            + [pltpu.SemaphoreType.DMA] * (3 * DEPTH)
        ),
    )
    def k(comb_hbm, z_hbm, out_hbm, *scr):
        cbuf = scr[0:DEPTH]
        sbuf = scr[DEPTH:2 * DEPTH]
        dbuf = scr[2 * DEPTH:3 * DEPTH]
        rows = scr[3 * DEPTH:4 * DEPTH]
        acc_sh = scr[4 * DEPTH]
        semc = scr[4 * DEPTH + 1:4 * DEPTH + 1 + DEPTH]
        semg = scr[4 * DEPTH + 1 + DEPTH:4 * DEPTH + 1 + 2 * DEPTH]
        sems = scr[4 * DEPTH + 1 + 2 * DEPTH:4 * DEPTH + 1 + 3 * DEPTH]

        cid = lax.axis_index("c")
        sid = lax.axis_index("s")
        wid = _worker_id()

        for e in range(K):
            for c in range(D // 16):
                rows[0][e, pl.ds(c * 16, 16)] = jnp.zeros((16,), f32)
        for j in range(RPT // K):
            pltpu.sync_copy(rows[0], acc_sh.at[pl.ds(sid * RPT + j * K, K)])
        plsc.subcore_barrier()

        def blk_slice(b):
            return comb_hbm.at[wid * NBLK + b]

        for d in range(DEPTH):
            pltpu.async_copy(blk_slice(d), cbuf[d], semc[d])

        def stage_front(i, d):
            b = i * DEPTH + d

            @pl.when(i > 0)
            def _ws():
                pltpu.make_async_copy(rows[d], acc_sh.at[dbuf[d]],
                                      sems[d]).wait()

            pltpu.make_async_copy(blk_slice(b), cbuf[d], semc[d]).wait()

            @pl.loop(0, K // 16)
            def _ix(j):
                sl = pl.ds(j * 16, 16)
                code = cbuf[d][sl]
                sbuf[d][sl] = code & 0x3FFF
                dbuf[d][sl] = (code >> 14) & 0x3FFF

            pltpu.async_copy(z_hbm.at[sbuf[d]], rows[d], semg[d])

            @pl.when(i < ROUNDS - 1)
            def _p():
                pltpu.async_copy(blk_slice(b + DEPTH), cbuf[d], semc[d])

        def stage_back(d):
            pltpu.make_async_copy(z_hbm.at[sbuf[d]], rows[d], semg[d]).wait()
            pltpu.async_copy(rows[d], acc_sh.at[dbuf[d]], sems[d], add=True)

        @pl.loop(0, ROUNDS)
        def _r(i):
            for d in range(DEPTH):
                stage_front(i, d)
            for d in range(DEPTH):
                stage_back(d)

        for d in range(DEPTH):
            pltpu.make_async_copy(rows[d], acc_sh.at[dbuf[d]], sems[d]).wait()
        plsc.subcore_barrier()
        for j in range(RPT // K):
            off = sid * RPT + j * K
            pltpu.sync_copy(acc_sh.at[pl.ds(off, K)], rows[0])
            pltpu.sync_copy(rows[0], out_hbm.at[pl.ds(cid * NPAD + off, K)])

    return k(comb, z)


def _weight_tc(comp, bases2d):
    def body(c_ref, b_ref, w_ref):
        w_ref[...] = jnp.dot(c_ref[...], b_ref[...],
                             preferred_element_type=f32)

    return pl.pallas_call(
        body,
        out_shape=jax.ShapeDtypeStruct((R, D * D), f32),
    )(comp, bases2d)


def _y_tc(x, w3):
    NT = 10
    BN = N // NT

    def body(x_ref, w_ref, y_ref):
        y_ref[0] = jnp.dot(x_ref[...], w_ref[0], preferred_element_type=f32)

    return pl.pallas_call(
        body,
        grid=(R, NT),
        in_specs=[
            pl.BlockSpec((BN, D), lambda r, n: (n, 0)),
            pl.BlockSpec((1, D, D), lambda r, n: (r, 0, 0)),
        ],
        out_specs=pl.BlockSpec((1, BN, D), lambda r, n: (r, n, 0)),
        out_shape=jax.ShapeDtypeStruct((R, N, D), f32),
    )(x, w3)


def _inv_tc(cnt):
    CW = CBINS // 8
    c2 = cnt.reshape(NC * 8, CW)

    def body(c_ref, o_ref):
        o_ref[...] = 1.0 / jnp.maximum(c_ref[0:8] + c_ref[8:16], 1.0)

    inv = pl.pallas_call(
        body,
        out_shape=jax.ShapeDtypeStruct((8, CW), f32),
    )(c2)
    return inv.reshape(CBINS)


def _fuse_tc(part, xin, w, bias, w2=None):
    NT = 10
    BN = N // NT
    specs = [
        pl.BlockSpec((NC, BN, D), lambda n: (0, n, 0)),
        pl.BlockSpec((BN, D), lambda n: (n, 0)),
        pl.BlockSpec((D, D), lambda n: (0, 0)),
        pl.BlockSpec((1, D), lambda n: (0, 0)),
    ]
    ospec = pl.BlockSpec((BN, D), lambda n: (n, 0))
    oshape = jax.ShapeDtypeStruct((N, D), f32)

    if w2 is None:
        def body(p_ref, x_ref, w_ref, b_ref, o_ref):
            o_ref[...] = (p_ref[0] + p_ref[1] + b_ref[...]
                          + jnp.dot(x_ref[...], w_ref[...],
                                    preferred_element_type=f32))

        return pl.pallas_call(
            body, grid=(NT,), in_specs=specs, out_specs=ospec,
            out_shape=oshape,
        )(part, xin, w, bias)

    def body(p_ref, x_ref, w_ref, b_ref, w2_ref, o_ref, z_ref):
        o = (p_ref[0] + p_ref[1] + b_ref[...]
             + jnp.dot(x_ref[...], w_ref[...], preferred_element_type=f32))
        o_ref[...] = o
        z_ref[...] = jnp.dot(o, w2_ref[...], preferred_element_type=f32)

    return pl.pallas_call(
        body, grid=(NT,),
        in_specs=specs + [pl.BlockSpec((D, D), lambda n: (0, 0))],
        out_specs=(ospec, ospec),
        out_shape=(oshape, oshape),
    )(part, xin, w, bias, w2)


def kernel(node_features, edge_index, edge_norm, edge_type, comp, bases,
           root1, bias1, w_rel, w_root, bias2):
    del edge_norm
    x = node_features
    src = edge_index[0]
    dst = edge_index[1]
    pad = EP - E
    src_p = jnp.concatenate([src, jnp.zeros((pad,), i32)])
    dst_p = jnp.concatenate([dst, jnp.full((pad,), N, i32)])
    et_p = jnp.concatenate([edge_type, jnp.zeros((pad,), i32)])
    comb = ((et_p << 28) | (dst_p << 14) | src_p).reshape(G, K)

    w2d = _weight_tc(comp, bases.reshape(NBASES, D * D))
    y = _y_tc(x, w2d.reshape(R, D, D)).reshape(R * N, D)

    inv = _inv_tc(_count_sc(comb))

    outp = _rgcn_sc(comb, inv, y)
    outp2 = outp.reshape(NC, NPAD, D)[:, :N]
    x1, z = _fuse_tc(outp2, x, root1, bias1.reshape(1, D), w2=w_rel)

    aggp = _gconv_sc(comb, z)
    aggp2 = aggp.reshape(NC, NPAD, D)[:, :N]
    x2 = _fuse_tc(aggp2, x1, w_root, bias2.reshape(1, D))
    return x2


# speedup vs baseline: 9.0211x; 1.0172x over previous
import functools

import jax
import jax.numpy as jnp
from jax import lax
from jax.experimental import pallas as pl
from jax.experimental.pallas import tpu as pltpu
from jax.experimental.pallas import tpu_sc as plsc

N = 10000
E = 320000
D = 128
R = 8
NBASES = 30

NC = 2
NS = 16
NW = NC * NS
K = 128
DEPTH = 2
EPW = -(-E // (NW * K * DEPTH)) * K * DEPTH
EP = NW * EPW
NBLK = EPW // K
NHALF = NBLK // 2
ROUNDS = NBLK // DEPTH
G = EP // K

NPAD = 10240
RPT = NPAD // NS
CBINS = 80128
CPT = CBINS // NS

f32 = jnp.float32
i32 = jnp.int32


def _mesh():
    return plsc.VectorSubcoreMesh(
        core_axis_name="c", subcore_axis_name="s", num_cores=NC, num_subcores=NS
    )


def _worker_id():
    return lax.axis_index("s") * NC + lax.axis_index("c")


def _splat(vec16, lane):
    idx = jnp.full((16, 1), lane, i32)
    return lax.gather(
        vec16, idx,
        dimension_numbers=lax.GatherDimensionNumbers(
            offset_dims=(), collapsed_slice_dims=(0,), start_index_map=(0,)),
        slice_sizes=(1,),
        mode=lax.GatherScatterMode.PROMISE_IN_BOUNDS,
    )


def _count_sc(comb):
    @functools.partial(
        pl.kernel,
        out_type=jax.ShapeDtypeStruct((NC * CBINS,), f32),
        mesh=_mesh(),
        scratch_types=[
            pltpu.VMEM((K,), i32),
            pltpu.VMEM((K,), i32),
            pltpu.VMEM((K,), i32),
            pltpu.VMEM((K,), i32),
            pltpu.VMEM((K,), f32),
            pltpu.VMEM((CPT,), f32),
            pltpu.VMEM_SHARED((CBINS,), f32),
            pltpu.SemaphoreType.DMA,
            pltpu.SemaphoreType.DMA,
            pltpu.SemaphoreType.DMA,
            pltpu.SemaphoreType.DMA,
        ],
    )
    def k(comb_hbm, out_hbm, cbufA, cbufB, kbufA, kbufB, ones, zbuf, cnt_sh,
          semcA, semcB, semsA, semsB):
        cid = lax.axis_index("c")
        sid = lax.axis_index("s")
        wid = _worker_id()

        @pl.loop(0, CPT // 16)
        def _z(i):
            zbuf[pl.ds(i * 16, 16)] = jnp.zeros((16,), f32)

        @pl.loop(0, K // 16)
        def _o(i):
            ones[pl.ds(i * 16, 16)] = jnp.ones((16,), f32)

        pltpu.sync_copy(zbuf, cnt_sh.at[pl.ds(sid * CPT, CPT)])
        plsc.subcore_barrier()

        def blk_slice(b):
            return comb_hbm.at[wid * NBLK + b]

        pltpu.async_copy(blk_slice(0), cbufA, semcA)
        pltpu.async_copy(blk_slice(1), cbufB, semcB)

        @pl.loop(0, NHALF)
        def _i2(i2):
            for off, cbuf, kbuf, semc, sems in (
                    (0, cbufA, kbufA, semcA, semsA),
                    (1, cbufB, kbufB, semcB, semsB)):
                b = i2 * 2 + off
                pltpu.make_async_copy(blk_slice(b), cbuf, semc).wait()

                @pl.when(i2 > 0)
                def _w():
                    pltpu.make_async_copy(ones, cnt_sh.at[kbuf], sems).wait()

                @pl.loop(0, K // 16)
                def _c(i):
                    sl = pl.ds(i * 16, 16)
                    code = cbuf[sl]
                    kbuf[sl] = ((code >> 14) & 0x3FFF) * R + (code >> 28)

                @pl.when(i2 < NHALF - 1)
                def _p():
                    pltpu.async_copy(blk_slice(b + 2), cbuf, semc)

                pltpu.async_copy(ones, cnt_sh.at[kbuf], sems, add=True)

        pltpu.make_async_copy(ones, cnt_sh.at[kbufA], semsA).wait()
        pltpu.make_async_copy(ones, cnt_sh.at[kbufB], semsB).wait()
        plsc.subcore_barrier()
        pltpu.sync_copy(cnt_sh.at[pl.ds(sid * CPT, CPT)], zbuf)
        pltpu.sync_copy(zbuf, out_hbm.at[pl.ds(cid * CBINS + sid * CPT, CPT)])

    return k(comb)


def _rgcn_sc(comb, inv, yflat):
    @functools.partial(
        pl.kernel,
        out_type=jax.ShapeDtypeStruct((NC * NPAD, D), f32),
        mesh=_mesh(),
        scratch_types=(
            [pltpu.VMEM((K,), i32)] * DEPTH
            + [pltpu.VMEM((K,), i32)] * DEPTH
            + [pltpu.VMEM((K,), i32)] * DEPTH
            + [pltpu.VMEM((K,), i32)] * DEPTH
            + [pltpu.VMEM((K,), f32)] * DEPTH
            + [pltpu.VMEM((K, D), f32)] * DEPTH
            + [pltpu.VMEM_SHARED((NPAD, D), f32)]
            + [pltpu.SemaphoreType.DMA] * (3 * DEPTH)
        ),
    )
    def k(comb_hbm, inv_hbm, y_hbm, out_hbm, *scr):
        cbuf = scr[0:DEPTH]
        ibuf = scr[DEPTH:2 * DEPTH]
        kbuf = scr[2 * DEPTH:3 * DEPTH]
        dbuf = scr[3 * DEPTH:4 * DEPTH]
        scb = scr[4 * DEPTH:5 * DEPTH]
        rows = scr[5 * DEPTH:6 * DEPTH]
        acc_sh = scr[6 * DEPTH]
        semc = scr[6 * DEPTH + 1:6 * DEPTH + 1 + DEPTH]
        semg = scr[6 * DEPTH + 1 + DEPTH:6 * DEPTH + 1 + 2 * DEPTH]
        sems = scr[6 * DEPTH + 1 + 2 * DEPTH:6 * DEPTH + 1 + 3 * DEPTH]

        cid = lax.axis_index("c")
        sid = lax.axis_index("s")
        wid = _worker_id()

        for e in range(K):
            for c in range(D // 16):
                rows[0][e, pl.ds(c * 16, 16)] = jnp.zeros((16,), f32)
        for j in range(RPT // K):
            pltpu.sync_copy(rows[0], acc_sh.at[pl.ds(sid * RPT + j * K, K)])
        plsc.subcore_barrier()

        def blk_slice(b):
            return comb_hbm.at[wid * NBLK + b]

        for d in range(DEPTH):
            pltpu.async_copy(blk_slice(d), cbuf[d], semc[d])

        def stage_front(i, d):
            b = i * DEPTH + d

            @pl.when(i > 0)
            def _ws():
                pltpu.make_async_copy(rows[d], acc_sh.at[dbuf[d]],
                                      sems[d]).wait()

            pltpu.make_async_copy(blk_slice(b), cbuf[d], semc[d]).wait()

            @pl.loop(0, K // 16)
            def _ix(j):
                sl = pl.ds(j * 16, 16)
                code = cbuf[d][sl]
                s16 = code & 0x3FFF
                d16 = (code >> 14) & 0x3FFF
                t16 = code >> 28
                ibuf[d][sl] = t16 * N + s16
                kbuf[d][sl] = d16 * R + t16
                dbuf[d][sl] = d16

            pltpu.async_copy(inv_hbm.at[kbuf[d]], scb[d], semg[d])
            pltpu.async_copy(y_hbm.at[ibuf[d]], rows[d], semg[d])

            @pl.when(i < ROUNDS - 1)
            def _p():
                pltpu.async_copy(blk_slice(b + DEPTH), cbuf[d], semc[d])

        def stage_back(d):
            pltpu.make_async_copy(inv_hbm.at[kbuf[d]], scb[d], semg[d]).wait()
            pltpu.make_async_copy(y_hbm.at[ibuf[d]], rows[d], semg[d]).wait()
            for gch in range(K // 16):
                sc16 = scb[d][pl.ds(gch * 16, 16)]
                for l in range(16):
                    spl = _splat(sc16, l)
                    e = gch * 16 + l
                    for c in range(D // 16):
                        sl = pl.ds(c * 16, 16)
                        rows[d][e, sl] = rows[d][e, sl] * spl
            pltpu.async_copy(rows[d], acc_sh.at[dbuf[d]], sems[d], add=True)

        @pl.loop(0, ROUNDS)
        def _r(i):
            for d in range(DEPTH):
                stage_front(i, d)
            for d in range(DEPTH):
                stage_back(d)

        for d in range(DEPTH):
            pltpu.make_async_copy(rows[d], acc_sh.at[dbuf[d]], sems[d]).wait()
        plsc.subcore_barrier()
        for j in range(RPT // K):
            off = sid * RPT + j * K
            pltpu.sync_copy(acc_sh.at[pl.ds(off, K)], rows[0])
            pltpu.sync_copy(rows[0], out_hbm.at[pl.ds(cid * NPAD + off, K)])

    return k(comb, inv, yflat)


def _gconv_sc(comb, z):
    @functools.partial(
        pl.kernel,
        out_type=jax.ShapeDtypeStruct((NC * NPAD, D), f32),
        mesh=_mesh(),
        scratch_types=(
            [pltpu.VMEM((K,), i32)] * DEPTH
            + [pltpu.VMEM((K,), i32)] * DEPTH
            + [pltpu.VMEM((K,), i32)] * DEPTH
            + [pltpu.VMEM((K, D), f32)] * DEPTH
            + [pltpu.VMEM_SHARED((NPAD, D), f32)]
            + [pltpu.SemaphoreType.DMA] * (3 * DEPTH)
        ),
    )
    def k(comb_hbm, z_hbm, out_hbm, *scr):
        cbuf = scr[0:DEPTH]
        sbuf = scr[DEPTH:2 * DEPTH]
        dbuf = scr[2 * DEPTH:3 * DEPTH]
        rows = scr[3 * DEPTH:4 * DEPTH]
        acc_sh = scr[4 * DEPTH]
        semc = scr[4 * DEPTH + 1:4 * DEPTH + 1 + DEPTH]
        semg = scr[4 * DEPTH + 1 + DEPTH:4 * DEPTH + 1 + 2 * DEPTH]
        sems = scr[4 * DEPTH + 1 + 2 * DEPTH:4 * DEPTH + 1 + 3 * DEPTH]

        cid = lax.axis_index("c")
        sid = lax.axis_index("s")
        wid = _worker_id()

        for e in range(K):
            for c in range(D // 16):
                rows[0][e, pl.ds(c * 16, 16)] = jnp.zeros((16,), f32)
        for j in range(RPT // K):
            pltpu.sync_copy(rows[0], acc_sh.at[pl.ds(sid * RPT + j * K, K)])
        plsc.subcore_barrier()

        def blk_slice(b):
            return comb_hbm.at[wid * NBLK + b]

        for d in range(DEPTH):
            pltpu.async_copy(blk_slice(d), cbuf[d], semc[d])

        def stage_front(i, d):
            b = i * DEPTH + d

            @pl.when(i > 0)
            def _ws():
                pltpu.make_async_copy(rows[d], acc_sh.at[dbuf[d]],
                                      sems[d]).wait()

            pltpu.make_async_copy(blk_slice(b), cbuf[d], semc[d]).wait()

            @pl.loop(0, K // 16)
            def _ix(j):
                sl = pl.ds(j * 16, 16)
                code = cbuf[d][sl]
                sbuf[d][sl] = code & 0x3FFF
                dbuf[d][sl] = (code >> 14) & 0x3FFF

            pltpu.async_copy(z_hbm.at[sbuf[d]], rows[d], semg[d])

            @pl.when(i < ROUNDS - 1)
            def _p():
                pltpu.async_copy(blk_slice(b + DEPTH), cbuf[d], semc[d])

        def stage_back(d):
            pltpu.make_async_copy(z_hbm.at[sbuf[d]], rows[d], semg[d]).wait()
            pltpu.async_copy(rows[d], acc_sh.at[dbuf[d]], sems[d], add=True)

        @pl.loop(0, ROUNDS)
        def _r(i):
            for d in range(DEPTH):
                stage_front(i, d)
            for d in range(DEPTH):
                stage_back(d)

        for d in range(DEPTH):
            pltpu.make_async_copy(rows[d], acc_sh.at[dbuf[d]], sems[d]).wait()
        plsc.subcore_barrier()
        for j in range(RPT // K):
            off = sid * RPT + j * K
            pltpu.sync_copy(acc_sh.at[pl.ds(off, K)], rows[0])
            pltpu.sync_copy(rows[0], out_hbm.at[pl.ds(cid * NPAD + off, K)])

    return k(comb, z)


def _weight_tc(comp, bases2d):
    def body(c_ref, b_ref, w_ref):
        w_ref[...] = jnp.dot(c_ref[...], b_ref[...],
                             preferred_element_type=f32)

    return pl.pallas_call(
        body,
        out_shape=jax.ShapeDtypeStruct((R, D * D), f32),
    )(comp, bases2d)


def _y_tc(x, w3):
    NT = 10
    BN = N // NT

    def body(x_ref, w_ref, y_ref):
        y_ref[0] = jnp.dot(x_ref[...], w_ref[0], preferred_element_type=f32)

    return pl.pallas_call(
        body,
        grid=(R, NT),
        in_specs=[
            pl.BlockSpec((BN, D), lambda r, n: (n, 0)),
            pl.BlockSpec((1, D, D), lambda r, n: (r, 0, 0)),
        ],
        out_specs=pl.BlockSpec((1, BN, D), lambda r, n: (r, n, 0)),
        out_shape=jax.ShapeDtypeStruct((R, N, D), f32),
    )(x, w3)


def _inv_tc(cnt):
    CW = CBINS // 8
    c2 = cnt.reshape(NC * 8, CW)

    def body(c_ref, o_ref):
        o_ref[...] = 1.0 / jnp.maximum(c_ref[0:8] + c_ref[8:16], 1.0)

    inv = pl.pallas_call(
        body,
        out_shape=jax.ShapeDtypeStruct((8, CW), f32),
    )(c2)
    return inv.reshape(CBINS)


def _fuse_tc(part, xin, w, bias, w2=None):
    NT = 10
    BN = N // NT
    specs = [
        pl.BlockSpec((NC, BN, D), lambda n: (0, n, 0)),
        pl.BlockSpec((BN, D), lambda n: (n, 0)),
        pl.BlockSpec((D, D), lambda n: (0, 0)),
        pl.BlockSpec((1, D), lambda n: (0, 0)),
    ]
    ospec = pl.BlockSpec((BN, D), lambda n: (n, 0))
    oshape = jax.ShapeDtypeStruct((N, D), f32)

    if w2 is None:
        def body(p_ref, x_ref, w_ref, b_ref, o_ref):
            o_ref[...] = (p_ref[0] + p_ref[1] + b_ref[...]
                          + jnp.dot(x_ref[...], w_ref[...],
                                    preferred_element_type=f32))

        return pl.pallas_call(
            body, grid=(NT,), in_specs=specs, out_specs=ospec,
            out_shape=oshape,
        )(part, xin, w, bias)

    def body(p_ref, x_ref, w_ref, b_ref, w2_ref, o_ref, z_ref):
        o = (p_ref[0] + p_ref[1] + b_ref[...]
             + jnp.dot(x_ref[...], w_ref[...], preferred_element_type=f32))
        o_ref[...] = o
        z_ref[...] = jnp.dot(o, w2_ref[...], preferred_element_type=f32)

    return pl.pallas_call(
        body, grid=(NT,),
        in_specs=specs + [pl.BlockSpec((D, D), lambda n: (0, 0))],
        out_specs=(ospec, ospec),
        out_shape=(oshape, oshape),
    )(part, xin, w, bias, w2)


def kernel(node_features, edge_index, edge_norm, edge_type, comp, bases,
           root1, bias1, w_rel, w_root, bias2):
    del edge_norm
    x = node_features
    src = edge_index[0]
    dst = edge_index[1]
    pad = EP - E
    src_p = jnp.concatenate([src, jnp.zeros((pad,), i32)])
    dst_p = jnp.concatenate([dst, jnp.full((pad,), N, i32)])
    et_p = jnp.concatenate([edge_type, jnp.zeros((pad,), i32)])
    comb = ((et_p << 28) | (dst_p << 14) | src_p).reshape(G, K)

    w2d = _weight_tc(comp, bases.reshape(NBASES, D * D))
    y = _y_tc(x, w2d.reshape(R, D, D)).reshape(R * N, D)

    inv = _inv_tc(_count_sc(comb))

    outp = _rgcn_sc(comb, inv, y)
    outp2 = outp.reshape(NC, NPAD, D)[:, :N]
    x1, z = _fuse_tc(outp2, x, root1, bias1.reshape(1, D), w2=w_rel)

    aggp = _gconv_sc(comb, z)
    aggp2 = aggp.reshape(NC, NPAD, D)[:, :N]
    x2 = _fuse_tc(aggp2, x1, w_root, bias2.reshape(1, D))
    return x2


# speedup vs baseline: 9.0236x; 1.0003x over previous
import functools

import jax
import jax.numpy as jnp
from jax import lax
from jax.experimental import pallas as pl
from jax.experimental.pallas import tpu as pltpu
from jax.experimental.pallas import tpu_sc as plsc

N = 10000
E = 320000
D = 128
R = 8
NBASES = 30

NC = 2
NS = 16
NW = NC * NS
K = 128
DEPTH = 2
EPW = -(-E // (NW * K * DEPTH)) * K * DEPTH
EP = NW * EPW
NBLK = EPW // K
NHALF = NBLK // 2
ROUNDS = NBLK // DEPTH
G = EP // K

NPAD = 10240
RPT = NPAD // NS
CBINS = 80128
CPT = CBINS // NS

f32 = jnp.float32
i32 = jnp.int32


def _mesh():
    return plsc.VectorSubcoreMesh(
        core_axis_name="c", subcore_axis_name="s", num_cores=NC, num_subcores=NS
    )


def _worker_id():
    return lax.axis_index("s") * NC + lax.axis_index("c")


def _splat(vec16, lane):
    idx = jnp.full((16, 1), lane, i32)
    return lax.gather(
        vec16, idx,
        dimension_numbers=lax.GatherDimensionNumbers(
            offset_dims=(), collapsed_slice_dims=(0,), start_index_map=(0,)),
        slice_sizes=(1,),
        mode=lax.GatherScatterMode.PROMISE_IN_BOUNDS,
    )


def _count_sc(comb):
    @functools.partial(
        pl.kernel,
        out_type=jax.ShapeDtypeStruct((NC * CBINS,), f32),
        mesh=_mesh(),
        scratch_types=[
            pltpu.VMEM((K,), i32),
            pltpu.VMEM((K,), i32),
            pltpu.VMEM((K,), i32),
            pltpu.VMEM((K,), i32),
            pltpu.VMEM((K,), f32),
            pltpu.VMEM((CPT,), f32),
            pltpu.VMEM_SHARED((CBINS,), f32),
            pltpu.SemaphoreType.DMA,
            pltpu.SemaphoreType.DMA,
            pltpu.SemaphoreType.DMA,
            pltpu.SemaphoreType.DMA,
        ],
    )
    def k(comb_hbm, out_hbm, cbufA, cbufB, kbufA, kbufB, ones, zbuf, cnt_sh,
          semcA, semcB, semsA, semsB):
        cid = lax.axis_index("c")
        sid = lax.axis_index("s")
        wid = _worker_id()

        @pl.loop(0, CPT // 16)
        def _z(i):
            zbuf[pl.ds(i * 16, 16)] = jnp.zeros((16,), f32)

        @pl.loop(0, K // 16)
        def _o(i):
            ones[pl.ds(i * 16, 16)] = jnp.ones((16,), f32)

        pltpu.sync_copy(zbuf, cnt_sh.at[pl.ds(sid * CPT, CPT)])
        plsc.subcore_barrier()

        def blk_slice(b):
            return comb_hbm.at[wid * NBLK + b]

        pltpu.async_copy(blk_slice(0), cbufA, semcA)
        pltpu.async_copy(blk_slice(1), cbufB, semcB)

        @pl.loop(0, NHALF)
        def _i2(i2):
            for off, cbuf, kbuf, semc, sems in (
                    (0, cbufA, kbufA, semcA, semsA),
                    (1, cbufB, kbufB, semcB, semsB)):
                b = i2 * 2 + off
                pltpu.make_async_copy(blk_slice(b), cbuf, semc).wait()

                @pl.when(i2 > 0)
                def _w():
                    pltpu.make_async_copy(ones, cnt_sh.at[kbuf], sems).wait()

                @pl.loop(0, K // 16)
                def _c(i):
                    sl = pl.ds(i * 16, 16)
                    code = cbuf[sl]
                    kbuf[sl] = ((code >> 14) & 0x3FFF) * R + (code >> 28)

                @pl.when(i2 < NHALF - 1)
                def _p():
                    pltpu.async_copy(blk_slice(b + 2), cbuf, semc)

                pltpu.async_copy(ones, cnt_sh.at[kbuf], sems, add=True)

        pltpu.make_async_copy(ones, cnt_sh.at[kbufA], semsA).wait()
        pltpu.make_async_copy(ones, cnt_sh.at[kbufB], semsB).wait()
        plsc.subcore_barrier()
        pltpu.sync_copy(cnt_sh.at[pl.ds(sid * CPT, CPT)], zbuf)
        pltpu.sync_copy(zbuf, out_hbm.at[pl.ds(cid * CBINS + sid * CPT, CPT)])

    return k(comb)


def _rgcn_sc(comb, inv, ypk):
    @functools.partial(
        pl.kernel,
        out_type=jax.ShapeDtypeStruct((NC * NPAD, D), f32),
        mesh=_mesh(),
        scratch_types=(
            [pltpu.VMEM((K,), i32)] * DEPTH
            + [pltpu.VMEM((K,), i32)] * DEPTH
            + [pltpu.VMEM((K,), i32)] * DEPTH
            + [pltpu.VMEM((K,), i32)] * DEPTH
            + [pltpu.VMEM((K,), f32)] * DEPTH
            + [pltpu.VMEM((K, D), f32)] * DEPTH
            + [pltpu.VMEM_SHARED((NPAD, D), f32)]
            + [pltpu.SemaphoreType.DMA] * (3 * DEPTH)
        ),
    )
    def k(comb_hbm, inv_hbm, y_hbm, out_hbm, *scr):
        cbuf = scr[0:DEPTH]
        ibuf = scr[DEPTH:2 * DEPTH]
        kbuf = scr[2 * DEPTH:3 * DEPTH]
        dbuf = scr[3 * DEPTH:4 * DEPTH]
        scb = scr[4 * DEPTH:5 * DEPTH]
        rows = scr[5 * DEPTH:6 * DEPTH]
        acc_sh = scr[6 * DEPTH]
        semc = scr[6 * DEPTH + 1:6 * DEPTH + 1 + DEPTH]
        semg = scr[6 * DEPTH + 1 + DEPTH:6 * DEPTH + 1 + 2 * DEPTH]
        sems = scr[6 * DEPTH + 1 + 2 * DEPTH:6 * DEPTH + 1 + 3 * DEPTH]

        cid = lax.axis_index("c")
        sid = lax.axis_index("s")
        wid = _worker_id()

        for e in range(K):
            for c in range(D // 16):
                rows[0][e, pl.ds(c * 16, 16)] = jnp.zeros((16,), f32)
        for j in range(RPT // K):
            pltpu.sync_copy(rows[0], acc_sh.at[pl.ds(sid * RPT + j * K, K)])
        plsc.subcore_barrier()

        def blk_slice(b):
            return comb_hbm.at[wid * NBLK + b]

        for d in range(DEPTH):
            pltpu.async_copy(blk_slice(d), cbuf[d], semc[d])

        def stage_front(i, d):
            b = i * DEPTH + d

            @pl.when(i > 0)
            def _ws():
                pltpu.make_async_copy(rows[d], acc_sh.at[dbuf[d]],
                                      sems[d]).wait()

            pltpu.make_async_copy(blk_slice(b), cbuf[d], semc[d]).wait()

            @pl.loop(0, K // 16)
            def _ix(j):
                sl = pl.ds(j * 16, 16)
                code = cbuf[d][sl]
                s16 = code & 0x3FFF
                d16 = (code >> 14) & 0x3FFF
                t16 = code >> 28
                ibuf[d][sl] = t16 * N + s16
                kbuf[d][sl] = d16 * R + t16
                dbuf[d][sl] = d16

            pltpu.async_copy(inv_hbm.at[kbuf[d]], scb[d], semg[d])
            pltpu.async_copy(y_hbm.at[ibuf[d]], rows[d], semg[d])

            @pl.when(i < ROUNDS - 1)
            def _p():
                pltpu.async_copy(blk_slice(b + DEPTH), cbuf[d], semc[d])

        def stage_back(d):
            pltpu.make_async_copy(inv_hbm.at[kbuf[d]], scb[d], semg[d]).wait()
            pltpu.make_async_copy(y_hbm.at[ibuf[d]], rows[d], semg[d]).wait()
            for gch in range(K // 16):
                sc16 = scb[d][pl.ds(gch * 16, 16)]
                for l in range(16):
                    spl = _splat(sc16, l)
                    e = gch * 16 + l
                    for g in range(D // 16):
                        sl = pl.ds(g * 16, 16)
                        rows[d][e, sl] = rows[d][e, sl] * spl
            pltpu.async_copy(rows[d], acc_sh.at[dbuf[d]], sems[d], add=True)

        @pl.loop(0, ROUNDS)
        def _r(i):
            for d in range(DEPTH):
                stage_front(i, d)
            for d in range(DEPTH):
                stage_back(d)

        for d in range(DEPTH):
            pltpu.make_async_copy(rows[d], acc_sh.at[dbuf[d]], sems[d]).wait()
        plsc.subcore_barrier()
        for j in range(RPT // K):
            off = sid * RPT + j * K
            pltpu.sync_copy(acc_sh.at[pl.ds(off, K)], rows[0])
            pltpu.sync_copy(rows[0], out_hbm.at[pl.ds(cid * NPAD + off, K)])

    return k(comb, inv, ypk)


def _gconv_sc(comb, z):
    @functools.partial(
        pl.kernel,
        out_type=jax.ShapeDtypeStruct((NC * NPAD, D), f32),
        mesh=_mesh(),
        scratch_types=(
            [pltpu.VMEM((K,), i32)] * DEPTH
            + [pltpu.VMEM((K,), i32)] * DEPTH
            + [pltpu.VMEM((K,), i32)] * DEPTH
            + [pltpu.VMEM((K, D), f32)] * DEPTH
            + [pltpu.VMEM_SHARED((NPAD, D), f32)]
            + [pltpu.SemaphoreType.DMA] * (3 * DEPTH)
        ),
    )
    def k(comb_hbm, z_hbm, out_hbm, *scr):
        cbuf = scr[0:DEPTH]
        sbuf = scr[DEPTH:2 * DEPTH]
        dbuf = scr[2 * DEPTH:3 * DEPTH]
        rows = scr[3 * DEPTH:4 * DEPTH]
        acc_sh = scr[4 * DEPTH]
        semc = scr[4 * DEPTH + 1:4 * DEPTH + 1 + DEPTH]
        semg = scr[4 * DEPTH + 1 + DEPTH:4 * DEPTH + 1 + 2 * DEPTH]
        sems = scr[4 * DEPTH + 1 + 2 * DEPTH:4 * DEPTH + 1 + 3 * DEPTH]

        cid = lax.axis_index("c")
        sid = lax.axis_index("s")
        wid = _worker_id()

        for e in range(K):
            for c in range(D // 16):
                rows[0][e, pl.ds(c * 16, 16)] = jnp.zeros((16,), f32)
        for j in range(RPT // K):
            pltpu.sync_copy(rows[0], acc_sh.at[pl.ds(sid * RPT + j * K, K)])
        plsc.subcore_barrier()

        def blk_slice(b):
            return comb_hbm.at[wid * NBLK + b]

        for d in range(DEPTH):
            pltpu.async_copy(blk_slice(d), cbuf[d], semc[d])

        def stage_front(i, d):
            b = i * DEPTH + d

            @pl.when(i > 0)
            def _ws():
                pltpu.make_async_copy(rows[d], acc_sh.at[dbuf[d]],
                                      sems[d]).wait()

            pltpu.make_async_copy(blk_slice(b), cbuf[d], semc[d]).wait()

            @pl.loop(0, K // 16)
            def _ix(j):
                sl = pl.ds(j * 16, 16)
                code = cbuf[d][sl]
                sbuf[d][sl] = code & 0x3FFF
                dbuf[d][sl] = (code >> 14) & 0x3FFF

            pltpu.async_copy(z_hbm.at[sbuf[d]], rows[d], semg[d])

            @pl.when(i < ROUNDS - 1)
            def _p():
                pltpu.async_copy(blk_slice(b + DEPTH), cbuf[d], semc[d])

        def stage_back(d):
            pltpu.make_async_copy(z_hbm.at[sbuf[d]], rows[d], semg[d]).wait()
            pltpu.async_copy(rows[d], acc_sh.at[dbuf[d]], sems[d], add=True)

        @pl.loop(0, ROUNDS)
        def _r(i):
            for d in range(DEPTH):
                stage_front(i, d)
            for d in range(DEPTH):
                stage_back(d)

        for d in range(DEPTH):
            pltpu.make_async_copy(rows[d], acc_sh.at[dbuf[d]], sems[d]).wait()
        plsc.subcore_barrier()
        for j in range(RPT // K):
            off = sid * RPT + j * K
            pltpu.sync_copy(acc_sh.at[pl.ds(off, K)], rows[0])
            pltpu.sync_copy(rows[0], out_hbm.at[pl.ds(cid * NPAD + off, K)])

    return k(comb, z)


def _weight_tc(comp, bases2d):
    def body(c_ref, b_ref, w_ref):
        w_ref[...] = jnp.dot(c_ref[...], b_ref[...],
                             preferred_element_type=f32)

    return pl.pallas_call(
        body,
        out_shape=jax.ShapeDtypeStruct((R, D * D), f32),
    )(comp, bases2d)


def _y_tc(x, w3):
    NT = 10
    BN = N // NT

    def body(x_ref, w_ref, y_ref):
        y_ref[0] = jnp.dot(x_ref[...], w_ref[0], preferred_element_type=f32)

    return pl.pallas_call(
        body,
        grid=(R, NT),
        in_specs=[
            pl.BlockSpec((BN, D), lambda r, n: (n, 0)),
            pl.BlockSpec((1, D, D), lambda r, n: (r, 0, 0)),
        ],
        out_specs=pl.BlockSpec((1, BN, D), lambda r, n: (r, n, 0)),
        out_shape=jax.ShapeDtypeStruct((R, N, D), f32),
    )(x, w3)


def _inv_tc(cnt):
    CW = CBINS // 8
    c2 = cnt.reshape(NC * 8, CW)

    def body(c_ref, o_ref):
        o_ref[...] = 1.0 / jnp.maximum(c_ref[0:8] + c_ref[8:16], 1.0)

    inv = pl.pallas_call(
        body,
        out_shape=jax.ShapeDtypeStruct((8, CW), f32),
    )(c2)
    return inv.reshape(CBINS)


def _fuse_tc(part, xin, w, bias, w2=None):
    NT = 10
    BN = N // NT
    specs = [
        pl.BlockSpec((NC, BN, D), lambda n: (0, n, 0)),
        pl.BlockSpec((BN, D), lambda n: (n, 0)),
        pl.BlockSpec((D, D), lambda n: (0, 0)),
        pl.BlockSpec((1, D), lambda n: (0, 0)),
    ]
    ospec = pl.BlockSpec((BN, D), lambda n: (n, 0))
    oshape = jax.ShapeDtypeStruct((N, D), f32)

    if w2 is None:
        def body(p_ref, x_ref, w_ref, b_ref, o_ref):
            o_ref[...] = (p_ref[0] + p_ref[1] + b_ref[...]
                          + jnp.dot(x_ref[...], w_ref[...],
                                    preferred_element_type=f32))

        return pl.pallas_call(
            body, grid=(NT,), in_specs=specs, out_specs=ospec,
            out_shape=oshape,
        )(part, xin, w, bias)

    def body(p_ref, x_ref, w_ref, b_ref, w2_ref, o_ref, z_ref):
        o = (p_ref[0] + p_ref[1] + b_ref[...]
             + jnp.dot(x_ref[...], w_ref[...], preferred_element_type=f32))
        o_ref[...] = o
        z_ref[...] = jnp.dot(o, w2_ref[...], preferred_element_type=f32)

    return pl.pallas_call(
        body, grid=(NT,),
        in_specs=specs + [pl.BlockSpec((D, D), lambda n: (0, 0))],
        out_specs=(ospec, ospec),
        out_shape=(oshape, oshape),
    )(part, xin, w, bias, w2)


def kernel(node_features, edge_index, edge_norm, edge_type, comp, bases,
           root1, bias1, w_rel, w_root, bias2):
    del edge_norm
    x = node_features
    src = edge_index[0]
    dst = edge_index[1]
    pad = EP - E
    src_p = jnp.concatenate([src, jnp.zeros((pad,), i32)])
    dst_p = jnp.concatenate([dst, jnp.full((pad,), N, i32)])
    et_p = jnp.concatenate([edge_type, jnp.zeros((pad,), i32)])
    comb = ((et_p << 28) | (dst_p << 14) | src_p).reshape(G, K)

    w2d = _weight_tc(comp, bases.reshape(NBASES, D * D))
    ypk = _y_tc(x, w2d.reshape(R, D, D)).reshape(R * N, D)

    inv = _inv_tc(_count_sc(comb))

    outp = _rgcn_sc(comb, inv, ypk)
    outp2 = outp.reshape(NC, NPAD, D)[:, :N]
    x1, z = _fuse_tc(outp2, x, root1, bias1.reshape(1, D), w2=w_rel)

    aggp = _gconv_sc(comb, z)
    aggp2 = aggp.reshape(NC, NPAD, D)[:, :N]
    x2 = _fuse_tc(aggp2, x1, w_root, bias2.reshape(1, D))
    return x2
